# Initial kernel scaffold; baseline (speedup 1.0000x reference)
import jax
import jax.numpy as jnp
from jax.experimental import pallas as pl


def kernel(input_ids, token_types, n_lower, n_upper, n_alpha, n_spaces, n_numeric, n_special, rx_ids, ry_ids, we_table, we_proj_w, we_proj_b, t_lower, t_upper, t_alpha, t_spaces, t_numeric, t_special, t_types, t_rx, t_ry):
    raise NotImplementedError("write your pallas kernel here")


# speedup vs baseline: 4.0206x; 4.0206x over previous
import functools

import jax
import jax.numpy as jnp
from jax import lax
from jax.experimental import pallas as pl
from jax.experimental.pallas import tpu as pltpu
from jax.experimental.pallas import tpu_sc as plsc

B, L = 1024, 200
TOK = B * L
D = 64
NC, NS = 2, 16
NW = NC * NS
TPW = TOK // NW
CHUNK = 128
NCHUNK = TPW // CHUNK


def _project_table(we_table, we_proj_w, we_proj_b):
    V, K = we_table.shape
    Dp = we_proj_w.shape[1]
    BM = 2048

    def mm(x_ref, w_ref, b_ref, o_ref):
        o_ref[...] = (
            jnp.dot(x_ref[...], w_ref[...], preferred_element_type=jnp.float32)
            + b_ref[...]
        )

    return pl.pallas_call(
        mm,
        grid=(pl.cdiv(V, BM),),
        in_specs=[
            pl.BlockSpec((BM, K), lambda i: (i, 0)),
            pl.BlockSpec((K, Dp), lambda i: (0, 0)),
            pl.BlockSpec((1, Dp), lambda i: (0, 0)),
        ],
        out_specs=pl.BlockSpec((BM, Dp), lambda i: (i, 0)),
        out_shape=jax.ShapeDtypeStruct((V, Dp), jnp.float32),
    )(we_table, we_proj_w, we_proj_b.reshape(1, Dp))


def _gather_sum(idx_stack, proj, t_types, t_lower, t_upper, t_alpha,
                t_spaces, t_numeric, t_special, t_rx, t_ry):
    mesh = plsc.VectorSubcoreMesh(core_axis_name="c", subcore_axis_name="s")

    @functools.partial(
        pl.kernel,
        out_type=jax.ShapeDtypeStruct((TOK, D), jnp.float32),
        mesh=mesh,
        scratch_types=[
            pltpu.VMEM((10, CHUNK), jnp.int32),
            pltpu.VMEM((CHUNK, D), jnp.float32),
            pltpu.SemaphoreType.DMA,
        ],
        compiler_params=pltpu.CompilerParams(use_tc_tiling_on_sc=False),
    )
    def sc(idx_hbm, proj_hbm, tt_hbm, tlo_hbm, tup_hbm, tal_hbm, tsp_hbm,
           tnu_hbm, tse_hbm, trx_hbm, try_hbm, out_hbm, idx_v, acc_v, sem):
        wid = lax.axis_index("c") * NS + lax.axis_index("s")
        base = wid * TPW
        smalls = (tt_hbm, tlo_hbm, tup_hbm, tal_hbm, tsp_hbm,
                  tnu_hbm, tse_hbm, trx_hbm, try_hbm)

        @pl.loop(0, NCHUNK)
        def _chunk(c):
            off = base + c * CHUNK
            pltpu.sync_copy(idx_hbm.at[:, pl.ds(off, CHUNK)], idx_v)
            pltpu.async_copy(proj_hbm.at[idx_v.at[0]], acc_v, sem).wait()
            for t, tab in enumerate(smalls):
                pltpu.async_copy(tab.at[idx_v.at[t + 1]], acc_v, sem,
                                 add=True).wait()
            pltpu.sync_copy(acc_v, out_hbm.at[pl.ds(off, CHUNK), :])

    return sc(idx_stack, proj, t_types, t_lower, t_upper, t_alpha,
              t_spaces, t_numeric, t_special, t_rx, t_ry)


def kernel(input_ids, token_types, n_lower, n_upper, n_alpha, n_spaces,
           n_numeric, n_special, rx_ids, ry_ids,
           we_table, we_proj_w, we_proj_b,
           t_lower, t_upper, t_alpha, t_spaces, t_numeric, t_special,
           t_types, t_rx, t_ry):
    proj = _project_table(we_table, we_proj_w, we_proj_b)
    idx_stack = jnp.stack(
        [input_ids.reshape(-1), token_types.reshape(-1),
         n_lower.reshape(-1), n_upper.reshape(-1), n_alpha.reshape(-1),
         n_spaces.reshape(-1), n_numeric.reshape(-1), n_special.reshape(-1),
         rx_ids.reshape(-1), ry_ids.reshape(-1)],
        axis=0).astype(jnp.int32)
    out = _gather_sum(idx_stack, proj, t_types, t_lower, t_upper, t_alpha,
                      t_spaces, t_numeric, t_special, t_rx, t_ry)
    return out.reshape(B, L, D)


# speedup vs baseline: 4.0426x; 1.0055x over previous
import functools

import jax
import jax.numpy as jnp
from jax import lax
from jax.experimental import pallas as pl
from jax.experimental.pallas import tpu as pltpu
from jax.experimental.pallas import tpu_sc as plsc

B, L = 1024, 200
TOK = B * L
D = 64
NC, NS = 2, 16
NW = NC * NS
TPW = TOK // NW
CHUNK = 128
NCHUNK = TPW // CHUNK


def _project_table(we_table, we_proj_w, we_proj_b):
    V, K = we_table.shape
    Dp = we_proj_w.shape[1]
    BM = 2048

    def mm(x_ref, w_ref, b_ref, o_ref):
        o_ref[...] = (
            jnp.dot(x_ref[...], w_ref[...], preferred_element_type=jnp.float32)
            + b_ref[...]
        )

    return pl.pallas_call(
        mm,
        grid=(pl.cdiv(V, BM),),
        in_specs=[
            pl.BlockSpec((BM, K), lambda i: (i, 0)),
            pl.BlockSpec((K, Dp), lambda i: (0, 0)),
            pl.BlockSpec((1, Dp), lambda i: (0, 0)),
        ],
        out_specs=pl.BlockSpec((BM, Dp), lambda i: (i, 0)),
        out_shape=jax.ShapeDtypeStruct((V, Dp), jnp.float32),
    )(we_table, we_proj_w, we_proj_b.reshape(1, Dp))


def _gather_sum(idx_stack, proj, t_types, t_lower, t_upper, t_alpha,
                t_spaces, t_numeric, t_special, t_rx, t_ry):
    mesh = plsc.VectorSubcoreMesh(core_axis_name="c", subcore_axis_name="s")

    @functools.partial(
        pl.kernel,
        out_type=jax.ShapeDtypeStruct((TOK, D), jnp.float32),
        mesh=mesh,
        scratch_types=[
            pltpu.VMEM((2, 10, CHUNK), jnp.int32),
            pltpu.VMEM((2, CHUNK, D), jnp.float32),
            pltpu.SemaphoreType.DMA((2,)),
            pltpu.SemaphoreType.DMA((2,)),
            pltpu.SemaphoreType.DMA((2,)),
        ],
        compiler_params=pltpu.CompilerParams(use_tc_tiling_on_sc=False),
    )
    def sc(idx_hbm, proj_hbm, tt_hbm, tlo_hbm, tup_hbm, tal_hbm, tsp_hbm,
           tnu_hbm, tse_hbm, trx_hbm, try_hbm, out_hbm,
           idx_v, acc_v, sem_idx, sem_g, sem_out):
        wid = lax.axis_index("c") * NS + lax.axis_index("s")
        base = wid * TPW
        tables = (proj_hbm, tt_hbm, tlo_hbm, tup_hbm, tal_hbm, tsp_hbm,
                  tnu_hbm, tse_hbm, trx_hbm, try_hbm)

        def fire_idx(c, p):
            off = base + c * CHUNK
            pltpu.async_copy(idx_hbm.at[:, pl.ds(off, CHUNK)],
                             idx_v.at[p], sem_idx.at[p])

        def wait_idx(p):
            pltpu.make_async_copy(idx_hbm.at[:, pl.ds(base, CHUNK)],
                                  idx_v.at[p], sem_idx.at[p]).wait()

        def zero_acc(p):
            z = jnp.zeros((16,), jnp.float32)

            @pl.loop(0, CHUNK, unroll=8)
            def _row(i):
                for j in range(D // 16):
                    acc_v[p, i, pl.ds(j * 16, 16)] = z

        def fire_gathers(c, p):
            for t, tab in enumerate(tables):
                pltpu.async_copy(tab.at[idx_v.at[p, t]], acc_v.at[p],
                                 sem_g.at[p], add=True)

        def drain_gathers(p):
            for t, tab in enumerate(tables):
                pltpu.make_async_copy(tab.at[idx_v.at[p, t]], acc_v.at[p],
                                      sem_g.at[p]).wait()

        def fire_wb(c, p):
            off = base + c * CHUNK
            pltpu.async_copy(acc_v.at[p], out_hbm.at[pl.ds(off, CHUNK), :],
                             sem_out.at[p])

        def drain_wb(p):
            pltpu.make_async_copy(acc_v.at[p], out_hbm.at[pl.ds(base, CHUNK), :],
                                  sem_out.at[p]).wait()

        fire_idx(0, 0)

        @pl.loop(0, NCHUNK, step=2)
        def _round(cbase):
            for p in (0, 1):
                c = cbase + p
                q = p ^ 1

                @pl.when(c >= 2)
                def _():
                    drain_wb(p)

                zero_acc(p)
                wait_idx(p)
                fire_gathers(c, p)

                @pl.when(c >= 1)
                def _():
                    drain_gathers(q)
                    fire_wb(c - 1, q)

                fire_idx(jnp.minimum(c + 1, NCHUNK - 1), q)

        drain_gathers(1)
        fire_wb(NCHUNK - 1, 1)
        wait_idx(0)
        drain_wb(0)
        drain_wb(1)

    return sc(idx_stack, proj, t_types, t_lower, t_upper, t_alpha,
              t_spaces, t_numeric, t_special, t_rx, t_ry)


def kernel(input_ids, token_types, n_lower, n_upper, n_alpha, n_spaces,
           n_numeric, n_special, rx_ids, ry_ids,
           we_table, we_proj_w, we_proj_b,
           t_lower, t_upper, t_alpha, t_spaces, t_numeric, t_special,
           t_types, t_rx, t_ry):
    proj = _project_table(we_table, we_proj_w, we_proj_b)
    idx_stack = jnp.stack(
        [input_ids.reshape(-1), token_types.reshape(-1),
         n_lower.reshape(-1), n_upper.reshape(-1), n_alpha.reshape(-1),
         n_spaces.reshape(-1), n_numeric.reshape(-1), n_special.reshape(-1),
         rx_ids.reshape(-1), ry_ids.reshape(-1)],
        axis=0).astype(jnp.int32)
    out = _gather_sum(idx_stack, proj, t_types, t_lower, t_upper, t_alpha,
                      t_spaces, t_numeric, t_special, t_rx, t_ry)
    return out.reshape(B, L, D)


# speedup vs baseline: 12.9327x; 3.1991x over previous
import functools

import jax
import jax.numpy as jnp
from jax import lax
from jax.experimental import pallas as pl
from jax.experimental.pallas import tpu as pltpu
from jax.experimental.pallas import tpu_sc as plsc

B, L = 1024, 200
TOK = B * L
D = 64
NC, NS = 2, 16
NW = NC * NS
TPW = TOK // NW
CHUNK = 128
NCHUNK = TPW // CHUNK


def _project_table(we_table, we_proj_w, we_proj_b):
    V, K = we_table.shape
    Dp = we_proj_w.shape[1]
    BM = 2048

    def mm(x_ref, w_ref, b_ref, o_ref):
        o_ref[...] = (
            jnp.dot(x_ref[...], w_ref[...], preferred_element_type=jnp.float32)
            + b_ref[...]
        )

    return pl.pallas_call(
        mm,
        grid=(pl.cdiv(V, BM),),
        in_specs=[
            pl.BlockSpec((BM, K), lambda i: (i, 0)),
            pl.BlockSpec((K, Dp), lambda i: (0, 0)),
            pl.BlockSpec((1, Dp), lambda i: (0, 0)),
        ],
        out_specs=pl.BlockSpec((BM, Dp), lambda i: (i, 0)),
        out_shape=jax.ShapeDtypeStruct((V, Dp), jnp.float32),
    )(we_table, we_proj_w, we_proj_b.reshape(1, Dp))


def _gather_sum(idx_stack, proj, rep):
    mesh = plsc.VectorSubcoreMesh(core_axis_name="c", subcore_axis_name="s")

    @functools.partial(
        pl.kernel,
        out_type=jax.ShapeDtypeStruct((TOK, D), jnp.float32),
        mesh=mesh,
        scratch_types=[
            pltpu.VMEM((2, 10, CHUNK), jnp.int32),
            pltpu.VMEM((2, CHUNK, D), jnp.float32),
            pltpu.SemaphoreType.DMA((2,)),
            pltpu.SemaphoreType.DMA((2,)),
            pltpu.SemaphoreType.DMA((2,)),
        ],
        compiler_params=pltpu.CompilerParams(use_tc_tiling_on_sc=False),
    )
    def sc(idx_hbm, proj_hbm, rep_hbm, out_hbm,
           idx_v, acc_v, sem_idx, sem_g, sem_out):
        wid = lax.axis_index("c") * NS + lax.axis_index("s")
        base = wid * TPW
        my_rep = rep_hbm.at[wid]
        tables = (proj_hbm,) + (my_rep,) * 9

        def fire_idx(c, p):
            off = base + c * CHUNK
            pltpu.async_copy(idx_hbm.at[:, pl.ds(off, CHUNK)],
                             idx_v.at[p], sem_idx.at[p])

        def wait_idx(p):
            pltpu.make_async_copy(idx_hbm.at[:, pl.ds(base, CHUNK)],
                                  idx_v.at[p], sem_idx.at[p]).wait()

        def zero_acc(p):
            z = jnp.zeros((16,), jnp.float32)

            @pl.loop(0, CHUNK, unroll=8)
            def _row(i):
                for j in range(D // 16):
                    acc_v[p, i, pl.ds(j * 16, 16)] = z

        def fire_gathers(c, p):
            for t, tab in enumerate(tables):
                pltpu.async_copy(tab.at[idx_v.at[p, t]], acc_v.at[p],
                                 sem_g.at[p], add=True)

        def drain_gathers(p):
            for t, tab in enumerate(tables):
                pltpu.make_async_copy(tab.at[idx_v.at[p, t]], acc_v.at[p],
                                      sem_g.at[p]).wait()

        def fire_wb(c, p):
            off = base + c * CHUNK
            pltpu.async_copy(acc_v.at[p], out_hbm.at[pl.ds(off, CHUNK), :],
                             sem_out.at[p])

        def drain_wb(p):
            pltpu.make_async_copy(acc_v.at[p], out_hbm.at[pl.ds(base, CHUNK), :],
                                  sem_out.at[p]).wait()

        fire_idx(0, 0)

        @pl.loop(0, NCHUNK, step=2)
        def _round(cbase):
            for p in (0, 1):
                c = cbase + p
                q = p ^ 1

                @pl.when(c >= 2)
                def _():
                    drain_wb(p)

                zero_acc(p)
                wait_idx(p)
                fire_gathers(c, p)

                @pl.when(c >= 1)
                def _():
                    drain_gathers(q)
                    fire_wb(c - 1, q)

                fire_idx(jnp.minimum(c + 1, NCHUNK - 1), q)

        drain_gathers(1)
        fire_wb(NCHUNK - 1, 1)
        wait_idx(0)
        drain_wb(0)
        drain_wb(1)

    return sc(idx_stack, proj, rep)


def kernel(input_ids, token_types, n_lower, n_upper, n_alpha, n_spaces,
           n_numeric, n_special, rx_ids, ry_ids,
           we_table, we_proj_w, we_proj_b,
           t_lower, t_upper, t_alpha, t_spaces, t_numeric, t_special,
           t_types, t_rx, t_ry):
    proj = _project_table(we_table, we_proj_w, we_proj_b)
    stacked = jnp.concatenate(
        [t_types, t_lower, t_upper, t_alpha, t_spaces, t_numeric, t_special,
         t_rx, t_ry], axis=0)
    rep = jnp.broadcast_to(stacked, (NW,) + stacked.shape)
    offs = [0, 4, 1004, 2004, 3004, 4004, 5004, 6004, 7004]
    idx_stack = jnp.stack(
        [input_ids.reshape(-1)]
        + [i.reshape(-1) + o for i, o in zip(
            [token_types, n_lower, n_upper, n_alpha, n_spaces, n_numeric,
             n_special, rx_ids, ry_ids], offs)],
        axis=0).astype(jnp.int32)
    out = _gather_sum(idx_stack, proj, rep)
    return out.reshape(B, L, D)
